# baseline (speedup 1.0000x reference)
import jax
import jax.numpy as jnp
from jax import lax
from jax.experimental import pallas as pl
from jax.experimental.pallas import tpu as pltpu
from jax.experimental.pallas import tpu_sc as plsc

HIDDIM = 128
N_NODES = 10000
N_EDGES = 320000
N_TUPLES = 320000

NC = 2
NS = 16
NW = NC * NS

CHUNK = 80
NBUF = 5
NA = N_EDGES // NW // CHUNK
NX = NA
NXC = 5
X_WORKERS = N_NODES // (NXC * CHUNK)


def _sc_body(x_hbm, a_hbm, t_hbm, x_table_hbm, ea_table_hbm,
             tuple_table_hbm, x_out, a_out, t_out,
             idx_x, idx_a, idx_t, rows0, rows1, rows2, rows3, rows4,
             xtab_s, etab_s, ttab_s,
             gsem0, gsem1, gsem2, gsem3, gsem4,
             ssem0, ssem1, ssem2, ssem3, ssem4,
             isem_x, isem_t):
    wid = lax.axis_index("s") * NC + lax.axis_index("c")
    rows = (rows0, rows1, rows2, rows3, rows4)
    gsems = (gsem0, gsem1, gsem2, gsem3, gsem4)
    ssems = (ssem0, ssem1, ssem2, ssem3, ssem4)
    e_per_w = N_EDGES // NW
    a_base = wid * e_per_w
    x_base = wid * NXC * CHUNK
    carries_x = wid < X_WORKERS

    t_idx_copy = pltpu.make_async_copy(
        t_hbm.at[pl.ds(a_base, e_per_w)], idx_t, isem_t)
    t_idx_copy.start()
    x_idx_copy = pltpu.make_async_copy(
        x_hbm.at[pl.ds(x_base, NXC * CHUNK)], idx_x, isem_x)

    @pl.when(carries_x)
    def _():
        x_idx_copy.start()

    @pl.when(lax.axis_index("s") == 0)
    def _():
        pltpu.sync_copy(x_table_hbm, xtab_s)
        pltpu.sync_copy(ea_table_hbm, etab_s)
        pltpu.sync_copy(tuple_table_hbm, ttab_s)

    pltpu.sync_copy(a_hbm.at[pl.ds(a_base, e_per_w)], idx_a)
    plsc.subcore_barrier()


    def gather_a(j, b):
        pltpu.async_copy(etab_s.at[idx_a.at[pl.ds(j * CHUNK, CHUNK)]],
                         rows[b], gsems[b])

    def gather_t(j, b):
        pltpu.async_copy(ttab_s.at[idx_t.at[pl.ds(j * CHUNK, CHUNK)]],
                         rows[b], gsems[b])

    def gather_x(j, b):
        pltpu.async_copy(xtab_s.at[idx_x.at[pl.ds(j * CHUNK, CHUNK)]],
                         rows[b], gsems[b])

    def gather_wait(b):
        pltpu.make_async_copy(a_out.at[pl.ds(0, CHUNK)], rows[b],
                              gsems[b]).wait()

    def scatter_a(j, b):
        pltpu.async_copy(rows[b],
                         a_out.at[pl.ds(a_base + j * CHUNK, CHUNK)],
                         ssems[b])

    def scatter_t(j, b):
        pltpu.async_copy(rows[b],
                         t_out.at[pl.ds(a_base + j * CHUNK, CHUNK)],
                         ssems[b])

    def scatter_x(j, b):
        pltpu.async_copy(rows[b],
                         x_out.at[pl.ds(x_base + j * CHUNK, CHUNK)],
                         ssems[b])

    def scatter_wait(b):
        pltpu.make_async_copy(rows[b], a_out.at[pl.ds(0, CHUNK)],
                              ssems[b]).wait()

    def scatter_any(ci, b):
        @pl.when(ci < NA)
        def _():
            scatter_a(ci, b)

        @pl.when(ci >= NA)
        def _():
            scatter_t(ci - NA, b)

    def gather_any(ci, b):
        @pl.when(ci < NA)
        def _():
            gather_a(ci, b)

        @pl.when(ci >= NA)
        def _():
            gather_t(ci - NA, b)

    gather_a(0, 0)
    gather_a(1, 1)
    t_idx_copy.wait()

    @pl.when(carries_x)
    def _():
        x_idx_copy.wait()

    gather_wait(0)
    scatter_a(0, 0)
    gather_a(2, 2)
    gather_wait(1)
    scatter_a(1, 1)
    gather_a(3, 3)
    gather_wait(2)
    scatter_a(2, 2)
    gather_a(4, 4)

    def body(g, carry):
        for k in range(NBUF):
            ci = 3 + NBUF * g + k
            b = (3 + k) % NBUF
            gather_wait(b)
            scatter_any(ci, b)
            scatter_wait((b + 2) % NBUF)
            gather_any(ci + 2, (b + 2) % NBUF)
        return carry

    lax.fori_loop(0, (NA + NX - 5) // NBUF, body, 0, unroll=False)

    gather_wait(3)
    scatter_t(NX - 2, 3)
    scatter_wait(0)

    @pl.when(carries_x)
    def _():
        gather_x(0, 0)

    gather_wait(4)
    scatter_t(NX - 1, 4)
    scatter_wait(1)

    @pl.when(carries_x)
    def _():
        gather_x(1, 1)

    @pl.when(carries_x)
    def _():
        gather_wait(0)
        scatter_x(0, 0)
        scatter_wait(2)
        gather_x(2, 2)
        gather_wait(1)
        scatter_x(1, 1)
        scatter_wait(3)
        gather_x(3, 3)
        gather_wait(2)
        scatter_x(2, 2)
        scatter_wait(4)
        gather_x(4, 4)
        gather_wait(3)
        scatter_x(3, 3)
        scatter_wait(0)
        gather_wait(4)
        scatter_x(4, 4)
        scatter_wait(1)

    scatter_wait(2)
    scatter_wait(3)
    scatter_wait(4)


@jax.jit
def _encode(x, A_values, X_values, x_table, ea_table, tuple_table):
    mesh = plsc.VectorSubcoreMesh(core_axis_name="c", subcore_axis_name="s")
    run = pl.kernel(
        _sc_body,
        out_type=(
            jax.ShapeDtypeStruct((N_NODES, HIDDIM), jnp.float32),
            jax.ShapeDtypeStruct((N_EDGES, HIDDIM), jnp.float32),
            jax.ShapeDtypeStruct((N_TUPLES, HIDDIM), jnp.float32),
        ),
        mesh=mesh,
        scratch_types=[
            pltpu.VMEM((NXC * CHUNK,), jnp.int32),
            pltpu.VMEM((N_EDGES // NW,), jnp.int32),
            pltpu.VMEM((N_TUPLES // NW,), jnp.int32),
            pltpu.VMEM((CHUNK, HIDDIM), jnp.float32),
            pltpu.VMEM((CHUNK, HIDDIM), jnp.float32),
            pltpu.VMEM((CHUNK, HIDDIM), jnp.float32),
            pltpu.VMEM((CHUNK, HIDDIM), jnp.float32),
            pltpu.VMEM((CHUNK, HIDDIM), jnp.float32),
            pltpu.MemorySpace.VMEM_SHARED((32, HIDDIM), jnp.float32),
            pltpu.MemorySpace.VMEM_SHARED((16, HIDDIM), jnp.float32),
            pltpu.MemorySpace.VMEM_SHARED((16, HIDDIM), jnp.float32),
            pltpu.SemaphoreType.DMA,
            pltpu.SemaphoreType.DMA,
            pltpu.SemaphoreType.DMA,
            pltpu.SemaphoreType.DMA,
            pltpu.SemaphoreType.DMA,
            pltpu.SemaphoreType.DMA,
            pltpu.SemaphoreType.DMA,
            pltpu.SemaphoreType.DMA,
            pltpu.SemaphoreType.DMA,
            pltpu.SemaphoreType.DMA,
            pltpu.SemaphoreType.DMA,
            pltpu.SemaphoreType.DMA,
        ],
    )
    return run(x, A_values, X_values, x_table, ea_table, tuple_table)


def kernel(x, A_values, X_values, x_table, ea_table, tuple_table):
    return _encode(x.astype(jnp.int32).reshape(-1), A_values, X_values,
                   x_table, ea_table, tuple_table)


# speedup vs baseline: 1.0037x; 1.0037x over previous
import jax
import jax.numpy as jnp
from jax import lax
from jax.experimental import pallas as pl
from jax.experimental.pallas import tpu as pltpu
from jax.experimental.pallas import tpu_sc as plsc

HIDDIM = 128
N_NODES = 10000
N_EDGES = 320000
N_TUPLES = 320000

NC = 2
NS = 16
NW = NC * NS

CHUNK = 80
NBUF = 6
NA = N_EDGES // NW // CHUNK
NX = NA
NXC = 5
X_WORKERS = N_NODES // (NXC * CHUNK)


def _sc_body(x_hbm, a_hbm, t_hbm, x_table_hbm, ea_table_hbm,
             tuple_table_hbm, x_out, a_out, t_out,
             idx_x, idx_a, idx_t,
             rows0, rows1, rows2, rows3, rows4, rows5,
             xtab_s, etab_s0, etab_s1, ttab_s0, ttab_s1,
             gsem0, gsem1, gsem2, gsem3, gsem4, gsem5,
             ssem0, ssem1, ssem2, ssem3, ssem4, ssem5,
             isem_x, isem_t):
    wid = lax.axis_index("s") * NC + lax.axis_index("c")
    rows = (rows0, rows1, rows2, rows3, rows4, rows5)
    gsems = (gsem0, gsem1, gsem2, gsem3, gsem4, gsem5)
    ssems = (ssem0, ssem1, ssem2, ssem3, ssem4, ssem5)
    etabs = (etab_s0, etab_s1)
    ttabs = (ttab_s0, ttab_s1)
    e_per_w = N_EDGES // NW
    a_base = wid * e_per_w
    x_base = wid * NXC * CHUNK
    carries_x = wid < X_WORKERS

    t_idx_copy = pltpu.make_async_copy(
        t_hbm.at[pl.ds(a_base, e_per_w)], idx_t, isem_t)
    t_idx_copy.start()
    x_idx_copy = pltpu.make_async_copy(
        x_hbm.at[pl.ds(x_base, NXC * CHUNK)], idx_x, isem_x)

    @pl.when(carries_x)
    def _():
        x_idx_copy.start()

    @pl.when(lax.axis_index("s") == 0)
    def _():
        pltpu.sync_copy(x_table_hbm, xtab_s)
        pltpu.sync_copy(ea_table_hbm, etab_s0)
        pltpu.sync_copy(tuple_table_hbm, ttab_s0)

    @pl.when(lax.axis_index("s") == 1)
    def _():
        pltpu.sync_copy(ea_table_hbm, etab_s1)
        pltpu.sync_copy(tuple_table_hbm, ttab_s1)

    pltpu.sync_copy(a_hbm.at[pl.ds(a_base, e_per_w)], idx_a)
    plsc.subcore_barrier()


    def gather_a(j, b, p):
        pltpu.async_copy(
            etabs[p].at[idx_a.at[pl.ds(j * CHUNK, CHUNK)]], rows[b],
            gsems[b])

    def gather_t(j, b, p):
        pltpu.async_copy(
            ttabs[p].at[idx_t.at[pl.ds(j * CHUNK, CHUNK)]], rows[b],
            gsems[b])

    def gather_x(j, b):
        pltpu.async_copy(xtab_s.at[idx_x.at[pl.ds(j * CHUNK, CHUNK)]],
                         rows[b], gsems[b])

    def gather_wait(b):
        pltpu.make_async_copy(a_out.at[pl.ds(0, CHUNK)], rows[b],
                              gsems[b]).wait()

    def scatter_a(j, b):
        pltpu.async_copy(rows[b],
                         a_out.at[pl.ds(a_base + j * CHUNK, CHUNK)],
                         ssems[b])

    def scatter_t(j, b):
        pltpu.async_copy(rows[b],
                         t_out.at[pl.ds(a_base + j * CHUNK, CHUNK)],
                         ssems[b])

    def scatter_x(j, b):
        pltpu.async_copy(rows[b],
                         x_out.at[pl.ds(x_base + j * CHUNK, CHUNK)],
                         ssems[b])

    def scatter_wait(b):
        pltpu.make_async_copy(rows[b], a_out.at[pl.ds(0, CHUNK)],
                              ssems[b]).wait()

    def scatter_any(ci, b):
        @pl.when(ci < NA)
        def _():
            scatter_a(ci, b)

        @pl.when(ci >= NA)
        def _():
            scatter_t(ci - NA, b)

    def gather_any(ci, b, p):
        @pl.when(ci < NA)
        def _():
            gather_a(ci, b, p)

        @pl.when(ci >= NA)
        def _():
            gather_t(ci - NA, b, p)

    gather_a(0, 0, 0)
    gather_a(1, 1, 1)
    gather_a(2, 2, 0)
    t_idx_copy.wait()

    @pl.when(carries_x)
    def _():
        x_idx_copy.wait()

    gather_wait(0)
    scatter_a(0, 0)
    gather_a(3, 3, 1)
    gather_wait(1)
    scatter_a(1, 1)
    gather_a(4, 4, 0)
    gather_wait(2)
    scatter_a(2, 2)
    gather_a(5, 5, 1)

    def body(g, carry):
        for k in range(NBUF):
            ci = 3 + NBUF * g + k
            b = (3 + k) % NBUF
            gather_wait(b)
            scatter_any(ci, b)
            scatter_wait((b + 3) % NBUF)
            gather_any(ci + 3, (b + 3) % NBUF, k & 1)
        return carry

    lax.fori_loop(0, (NA + NX - 10) // NBUF, body, 0, unroll=False)

    gather_wait(3)
    scatter_t(NX - 7, 3)
    scatter_wait(0)
    gather_t(NX - 4, 0, 0)
    gather_wait(4)
    scatter_t(NX - 6, 4)
    scatter_wait(1)
    gather_t(NX - 3, 1, 1)
    gather_wait(5)
    scatter_t(NX - 5, 5)
    scatter_wait(2)
    gather_t(NX - 2, 2, 0)
    gather_wait(0)
    scatter_t(NX - 4, 0)
    scatter_wait(3)
    gather_t(NX - 1, 3, 1)

    gather_wait(1)
    scatter_t(NX - 3, 1)
    scatter_wait(4)

    @pl.when(carries_x)
    def _():
        gather_x(0, 4)

    gather_wait(2)
    scatter_t(NX - 2, 2)
    scatter_wait(5)

    @pl.when(carries_x)
    def _():
        gather_x(1, 5)

    gather_wait(3)
    scatter_t(NX - 1, 3)
    scatter_wait(0)

    @pl.when(carries_x)
    def _():
        gather_x(2, 0)

    @pl.when(carries_x)
    def _():
        gather_wait(4)
        scatter_x(0, 4)
        scatter_wait(1)
        gather_x(3, 1)
        gather_wait(5)
        scatter_x(1, 5)
        scatter_wait(2)
        gather_x(4, 2)
        gather_wait(0)
        scatter_x(2, 0)
        scatter_wait(3)
        gather_wait(1)
        scatter_x(3, 1)
        scatter_wait(4)
        gather_wait(2)
        scatter_x(4, 2)
        scatter_wait(5)
        scatter_wait(0)

    scatter_wait(1)
    scatter_wait(2)

    @pl.when(jnp.logical_not(carries_x))
    def _():
        scatter_wait(3)


@jax.jit
def _encode(x, A_values, X_values, x_table, ea_table, tuple_table):
    mesh = plsc.VectorSubcoreMesh(core_axis_name="c", subcore_axis_name="s")
    run = pl.kernel(
        _sc_body,
        out_type=(
            jax.ShapeDtypeStruct((N_NODES, HIDDIM), jnp.float32),
            jax.ShapeDtypeStruct((N_EDGES, HIDDIM), jnp.float32),
            jax.ShapeDtypeStruct((N_TUPLES, HIDDIM), jnp.float32),
        ),
        mesh=mesh,
        scratch_types=[
            pltpu.VMEM((NXC * CHUNK,), jnp.int32),
            pltpu.VMEM((N_EDGES // NW,), jnp.int32),
            pltpu.VMEM((N_TUPLES // NW,), jnp.int32),
            pltpu.VMEM((CHUNK, HIDDIM), jnp.float32),
            pltpu.VMEM((CHUNK, HIDDIM), jnp.float32),
            pltpu.VMEM((CHUNK, HIDDIM), jnp.float32),
            pltpu.VMEM((CHUNK, HIDDIM), jnp.float32),
            pltpu.VMEM((CHUNK, HIDDIM), jnp.float32),
            pltpu.VMEM((CHUNK, HIDDIM), jnp.float32),
            pltpu.MemorySpace.VMEM_SHARED((32, HIDDIM), jnp.float32),
            pltpu.MemorySpace.VMEM_SHARED((16, HIDDIM), jnp.float32),
            pltpu.MemorySpace.VMEM_SHARED((16, HIDDIM), jnp.float32),
            pltpu.MemorySpace.VMEM_SHARED((16, HIDDIM), jnp.float32),
            pltpu.MemorySpace.VMEM_SHARED((16, HIDDIM), jnp.float32),
            pltpu.SemaphoreType.DMA,
            pltpu.SemaphoreType.DMA,
            pltpu.SemaphoreType.DMA,
            pltpu.SemaphoreType.DMA,
            pltpu.SemaphoreType.DMA,
            pltpu.SemaphoreType.DMA,
            pltpu.SemaphoreType.DMA,
            pltpu.SemaphoreType.DMA,
            pltpu.SemaphoreType.DMA,
            pltpu.SemaphoreType.DMA,
            pltpu.SemaphoreType.DMA,
            pltpu.SemaphoreType.DMA,
            pltpu.SemaphoreType.DMA,
            pltpu.SemaphoreType.DMA,
        ],
    )
    return run(x, A_values, X_values, x_table, ea_table, tuple_table)


def kernel(x, A_values, X_values, x_table, ea_table, tuple_table):
    return _encode(x.astype(jnp.int32).reshape(-1), A_values, X_values,
                   x_table, ea_table, tuple_table)
